# baseline (speedup 1.0000x reference)
import jax
import jax.numpy as jnp
from jax import lax
from jax.experimental import pallas as pl
from jax.experimental.pallas import tpu as pltpu

HIGHEST = lax.Precision.HIGHEST

STATE_DIM = 8
POLICY_DIM = 4
ATOM_NUM = 5
S2 = 24
S2P = 26
NP2 = S2 * S2
MAP_FULL_DIM = NP2 * 32


def _convs_kernel(x4_ref, w1_ref, b1_ref, w2_ref, b2_ref, o_ref,
                  p_ref, s2d_ref, sh1_ref, sh2_ref, imc_ref):
    BB = x4_ref.shape[0]
    M = BB * NP2

    for a in range(2):
        for b in range(2):
            p_ref[:, :, :, (2 * a + b) * 32:(2 * a + b + 1) * 32] = \
                x4_ref[:, a:a + 24, b:b + 24, :]

    acc1 = jnp.dot(p_ref[...].reshape(M, 128), w1_ref[...],
                   preferred_element_type=jnp.float32)
    m = jnp.maximum(jnp.maximum(acc1[:, 0:128], acc1[:, 128:256]),
                    jnp.maximum(acc1[:, 256:384], acc1[:, 384:512]))
    y1 = jnp.maximum(m + b1_ref[...], 0.0).astype(jnp.bfloat16)

    s2d_ref[:, :S2, :S2, :] = y1.reshape(BB, S2, S2, 128)
    s2d_ref[:, S2:, :, :] = jnp.zeros((BB, 2, S2P, 128), jnp.bfloat16)
    s2d_ref[:, :S2, S2:, :] = jnp.zeros((BB, S2, 2, 128), jnp.bfloat16)

    sh1_ref[...] = s2d_ref[:, :, 1:25, :]
    sh2_ref[...] = s2d_ref[:, :, 2:26, :]
    for qh in range(3):
        imc_ref[:, (3 * qh) * 128:(3 * qh + 1) * 128] = \
            s2d_ref[:, qh:qh + S2, 0:S2, :].reshape(M, 128)
        imc_ref[:, (3 * qh + 1) * 128:(3 * qh + 2) * 128] = \
            sh1_ref[:, qh:qh + S2, :, :].reshape(M, 128)
        imc_ref[:, (3 * qh + 2) * 128:(3 * qh + 3) * 128] = \
            sh2_ref[:, qh:qh + S2, :, :].reshape(M, 128)
    acc = jnp.dot(imc_ref[...], w2_ref[...],
                  preferred_element_type=jnp.float32)
    m2 = jnp.maximum(acc, pltpu.roll(acc, shift=64, axis=1))
    m2 = jnp.maximum(m2, pltpu.roll(m2, shift=32, axis=1))
    y2 = jnp.maximum(m2[:, :32] + b2_ref[...], 0.0)
    y2 = y2.astype(jnp.bfloat16).reshape(BB, NP2, 32)
    for j in range(4):
        o_ref[:, :, 32 * j:32 * (j + 1)] = y2[:, 144 * j:144 * (j + 1), :]


def _convs_call(s2d4, w1e, b1t, w2f, b2, BB):
    B = s2d4.shape[0]
    return pl.pallas_call(
        _convs_kernel,
        out_shape=jax.ShapeDtypeStruct((B, 144, 128), jnp.bfloat16),
        grid=(B // BB,),
        in_specs=[
            pl.BlockSpec((BB, 25, 25, 32), lambda i: (i, 0, 0, 0)),
            pl.BlockSpec((128, 512), lambda i: (0, 0)),
            pl.BlockSpec((1, 128), lambda i: (0, 0)),
            pl.BlockSpec((1152, 128), lambda i: (0, 0)),
            pl.BlockSpec((1, 32), lambda i: (0, 0)),
        ],
        out_specs=pl.BlockSpec((BB, 144, 128), lambda i: (i, 0, 0)),
        scratch_shapes=[
            pltpu.VMEM((BB, S2, S2, 128), jnp.bfloat16),
            pltpu.VMEM((BB, S2P, S2P, 128), jnp.bfloat16),
            pltpu.VMEM((BB, S2P, S2, 128), jnp.bfloat16),
            pltpu.VMEM((BB, S2P, S2, 128), jnp.bfloat16),
            pltpu.VMEM((BB * NP2, 9 * 128), jnp.bfloat16),
        ],
        compiler_params=pltpu.CompilerParams(
            dimension_semantics=("parallel",)),
    )(s2d4, w1e, b1t, w2f, b2)


def _heads_kernel(mapf_ref, st_ref, wmf_ref, bmf_ref, ws1_ref, bs1_ref,
                  ws2_ref, bs2_ref, wjs_ref, wjm_ref, bj_ref,
                  wq_ref, bq_ref, wsv_ref, bsv_ref, o_ref):
    def dot(a, b):
        return jnp.dot(a, b, precision=HIGHEST,
                       preferred_element_type=jnp.float32)

    map_lat = jnp.maximum(
        lax.dot_general(mapf_ref[...], wmf_ref[...],
                        (((1,), (1,)), ((), ())),
                        preferred_element_type=jnp.float32) + bmf_ref[...],
        0.0)
    h = jnp.maximum(dot(st_ref[...], ws1_ref[...]) + bs1_ref[...], 0.0)
    st_lat = jnp.maximum(dot(h, ws2_ref[...]) + bs2_ref[...], 0.0)
    joint = jnp.maximum(dot(st_lat, wjs_ref[...])
                        + dot(map_lat.astype(jnp.float32), wjm_ref[...])
                        + bj_ref[...], 0.0)
    q = dot(joint, wq_ref[...]) + bq_ref[...]
    sv = dot(joint, wsv_ref[...]) + bsv_ref[...]

    chunks = [q[:, a * ATOM_NUM:(a + 1) * ATOM_NUM] for a in range(POLICY_DIM)]
    qmean = sum(chunks) * (1.0 / POLICY_DIM)
    chunks = [sv + c - qmean for c in chunks]
    outs = []
    for z in chunks:
        mx = jnp.max(z, axis=-1, keepdims=True)
        lse = jnp.log(jnp.sum(jnp.exp(z - mx), axis=-1, keepdims=True)) + mx
        outs.append(z - lse)
    o_ref[...] = jnp.concatenate(outs, axis=-1)


def _heads_call(mapf, state, wmf, b_mf, w_s1, b_s1, w_s2, b_s2,
                w_js, w_jm, b_j, wq, bq, wsv, bsv, HB):
    B, K = mapf.shape
    pa = POLICY_DIM * ATOM_NUM
    return pl.pallas_call(
        _heads_kernel,
        out_shape=jax.ShapeDtypeStruct((B, pa), jnp.float32),
        grid=(B // HB,),
        in_specs=[
            pl.BlockSpec((HB, K), lambda i: (i, 0)),
            pl.BlockSpec((HB, STATE_DIM), lambda i: (i, 0)),
            pl.BlockSpec((50, K), lambda i: (0, 0)),
            pl.BlockSpec((1, 50), lambda i: (0, 0)),
            pl.BlockSpec((STATE_DIM, 64), lambda i: (0, 0)),
            pl.BlockSpec((1, 64), lambda i: (0, 0)),
            pl.BlockSpec((64, 50), lambda i: (0, 0)),
            pl.BlockSpec((1, 50), lambda i: (0, 0)),
            pl.BlockSpec((50, 50), lambda i: (0, 0)),
            pl.BlockSpec((50, 50), lambda i: (0, 0)),
            pl.BlockSpec((1, 50), lambda i: (0, 0)),
            pl.BlockSpec((50, pa), lambda i: (0, 0)),
            pl.BlockSpec((1, pa), lambda i: (0, 0)),
            pl.BlockSpec((50, ATOM_NUM), lambda i: (0, 0)),
            pl.BlockSpec((1, ATOM_NUM), lambda i: (0, 0)),
        ],
        out_specs=pl.BlockSpec((HB, pa), lambda i: (i, 0)),
        compiler_params=pltpu.CompilerParams(
            dimension_semantics=("parallel",)),
    )(mapf, state, wmf, b_mf[None, :], w_s1, b_s1[None, :],
      w_s2, b_s2[None, :], w_js, w_jm, b_j[None, :],
      wq, bq[None, :], wsv, bsv[None, :])


def _build_s2d4(x, B):
    pf = x[:, STATE_DIM:].reshape(B, 2, 25, 4, 25, 4).astype(jnp.bfloat16)
    return jnp.transpose(pf, (0, 2, 4, 3, 5, 1)).reshape(B, 25, 25, 32)


def _expand_w1(w1c):
    blk = w1c.reshape(3, 3, 8, 4, 32)
    parts = [jnp.pad(blk, ((rh, 1 - rh), (rw, 1 - rw), (0, 0), (0, 0), (0, 0)))
             for rh in range(2) for rw in range(2)]
    w1e = jnp.stack(parts, axis=4)
    w1e = w1e.reshape(128, 512)
    perm = []
    for al in range(2):
        for be in range(2):
            for dhp in range(2):
                for rh in range(2):
                    for dwp in range(2):
                        for rw in range(2):
                            for ic in range(2):
                                perm.append((2 * al + dhp) * 32
                                            + (2 * be + dwp) * 8
                                            + rh * 4 + rw * 2 + ic)
    w1e = w1e[jnp.asarray(perm), :]
    return w1e.astype(jnp.bfloat16)


def kernel(x, w1c, b1, w2c, b2, w_mf_t, b_mf, w_s1, b_s1, w_s2, b_s2,
           w_js, w_jm, b_j, wq, bq, wsv, bsv):
    B = x.shape[0]
    BB = next(bb for bb in (8, 4, 2, 1) if B % bb == 0)
    HB = B // 2 if B % 2 == 0 else B

    state = x[:, :STATE_DIM]
    s2d4 = _build_s2d4(x, B)
    w1e = _expand_w1(w1c)
    b1t = jnp.tile(b1, 4)[None, :]
    w2f = w2c.reshape(9 * 128, 128).astype(jnp.bfloat16)
    wmf_p = (w_mf_t.reshape(50, 4, 144, 32).transpose(0, 2, 1, 3)
             .reshape(50, MAP_FULL_DIM).astype(jnp.bfloat16))

    y2p = _convs_call(s2d4, w1e, b1t, w2f, b2[None, :], BB)
    mapf = y2p.reshape(B, MAP_FULL_DIM)
    out = _heads_call(mapf, state, wmf_p, b_mf,
                      w_s1, b_s1, w_s2, b_s2, w_js, w_jm, b_j,
                      wq, bq, wsv, bsv, HB)
    return out.reshape(B, POLICY_DIM, ATOM_NUM)


# speedup vs baseline: 1.0750x; 1.0750x over previous
import jax
import jax.numpy as jnp
from jax import lax
from jax.experimental import pallas as pl
from jax.experimental.pallas import tpu as pltpu

HIGHEST = lax.Precision.HIGHEST

STATE_DIM = 8
POLICY_DIM = 4
ATOM_NUM = 5
S2 = 24
S2P = 26
NP2 = S2 * S2
MAP_FULL_DIM = NP2 * 32


def _convs_kernel(x4_ref, w1_ref, b1_ref, w2_ref, b2_ref, o_ref,
                  p_ref, w3_ref):
    BB = x4_ref.shape[0]
    M = BB * NP2

    for a in range(2):
        for b in range(2):
            p_ref[:, :, :, (2 * a + b) * 32:(2 * a + b + 1) * 32] = \
                x4_ref[:, a:a + 24, b:b + 24, :]

    acc1 = jnp.dot(p_ref[...].reshape(M, 128), w1_ref[...],
                   preferred_element_type=jnp.float32)
    m = jnp.maximum(jnp.maximum(acc1[:, 0:128], acc1[:, 128:256]),
                    jnp.maximum(acc1[:, 256:384], acc1[:, 384:512]))
    y1 = jnp.maximum(m + b1_ref[...], 0.0).astype(jnp.bfloat16)
    y1r = y1.reshape(BB, S2, S2, 128)

    w3_ref[:, :S2, :, 0:128] = y1r
    w3_ref[:, :S2, 0:23, 128:256] = y1r[:, :, 1:24, :]
    w3_ref[:, :S2, 23:24, 128:256] = jnp.zeros((BB, S2, 1, 128), jnp.bfloat16)
    w3_ref[:, :S2, 0:22, 256:384] = y1r[:, :, 2:24, :]
    w3_ref[:, :S2, 22:24, 256:384] = jnp.zeros((BB, S2, 2, 128), jnp.bfloat16)
    w3_ref[:, S2:, :, :] = jnp.zeros((BB, 2, S2, 384), jnp.bfloat16)

    acc = None
    for qh in range(3):
        lhs = w3_ref[:, qh:qh + S2, :, :].reshape(M, 384)
        d = jnp.dot(lhs, w2_ref[qh], preferred_element_type=jnp.float32)
        acc = d if acc is None else acc + d
    m2 = jnp.maximum(acc, pltpu.roll(acc, shift=64, axis=1))
    m2 = jnp.maximum(m2, pltpu.roll(m2, shift=32, axis=1))
    y2 = jnp.maximum(m2[:, :32] + b2_ref[...], 0.0)
    y2 = y2.astype(jnp.bfloat16).reshape(BB, NP2, 32)
    for j in range(4):
        o_ref[:, :, 32 * j:32 * (j + 1)] = y2[:, 144 * j:144 * (j + 1), :]


def _convs_call(s2d4, w1e, b1t, w2f, b2, BB):
    B = s2d4.shape[0]
    return pl.pallas_call(
        _convs_kernel,
        out_shape=jax.ShapeDtypeStruct((B, 144, 128), jnp.bfloat16),
        grid=(B // BB,),
        in_specs=[
            pl.BlockSpec((BB, 25, 25, 32), lambda i: (i, 0, 0, 0)),
            pl.BlockSpec((128, 512), lambda i: (0, 0)),
            pl.BlockSpec((1, 128), lambda i: (0, 0)),
            pl.BlockSpec((3, 384, 128), lambda i: (0, 0, 0)),
            pl.BlockSpec((1, 32), lambda i: (0, 0)),
        ],
        out_specs=pl.BlockSpec((BB, 144, 128), lambda i: (i, 0, 0)),
        scratch_shapes=[
            pltpu.VMEM((BB, S2, S2, 128), jnp.bfloat16),
            pltpu.VMEM((BB, S2P, S2, 384), jnp.bfloat16),
        ],
        compiler_params=pltpu.CompilerParams(
            dimension_semantics=("parallel",)),
    )(s2d4, w1e, b1t, w2f, b2)


def _heads_kernel(mapf_ref, st_ref, wmf_ref, bmf_ref, ws1_ref, bs1_ref,
                  ws2_ref, bs2_ref, wjs_ref, wjm_ref, bj_ref,
                  wq_ref, bq_ref, wsv_ref, bsv_ref, o_ref):
    def dot(a, b):
        return jnp.dot(a, b, precision=HIGHEST,
                       preferred_element_type=jnp.float32)

    map_lat = jnp.maximum(
        lax.dot_general(mapf_ref[...], wmf_ref[...],
                        (((1,), (1,)), ((), ())),
                        preferred_element_type=jnp.float32) + bmf_ref[...],
        0.0)
    h = jnp.maximum(dot(st_ref[...], ws1_ref[...]) + bs1_ref[...], 0.0)
    st_lat = jnp.maximum(dot(h, ws2_ref[...]) + bs2_ref[...], 0.0)
    joint = jnp.maximum(dot(st_lat, wjs_ref[...])
                        + dot(map_lat.astype(jnp.float32), wjm_ref[...])
                        + bj_ref[...], 0.0)
    q = dot(joint, wq_ref[...]) + bq_ref[...]
    sv = dot(joint, wsv_ref[...]) + bsv_ref[...]

    chunks = [q[:, a * ATOM_NUM:(a + 1) * ATOM_NUM] for a in range(POLICY_DIM)]
    qmean = sum(chunks) * (1.0 / POLICY_DIM)
    chunks = [sv + c - qmean for c in chunks]
    outs = []
    for z in chunks:
        mx = jnp.max(z, axis=-1, keepdims=True)
        lse = jnp.log(jnp.sum(jnp.exp(z - mx), axis=-1, keepdims=True)) + mx
        outs.append(z - lse)
    o_ref[...] = jnp.concatenate(outs, axis=-1)


def _heads_call(mapf, state, wmf, b_mf, w_s1, b_s1, w_s2, b_s2,
                w_js, w_jm, b_j, wq, bq, wsv, bsv, HB):
    B, K = mapf.shape
    pa = POLICY_DIM * ATOM_NUM
    return pl.pallas_call(
        _heads_kernel,
        out_shape=jax.ShapeDtypeStruct((B, pa), jnp.float32),
        grid=(B // HB,),
        in_specs=[
            pl.BlockSpec((HB, K), lambda i: (i, 0)),
            pl.BlockSpec((HB, STATE_DIM), lambda i: (i, 0)),
            pl.BlockSpec((50, K), lambda i: (0, 0)),
            pl.BlockSpec((1, 50), lambda i: (0, 0)),
            pl.BlockSpec((STATE_DIM, 64), lambda i: (0, 0)),
            pl.BlockSpec((1, 64), lambda i: (0, 0)),
            pl.BlockSpec((64, 50), lambda i: (0, 0)),
            pl.BlockSpec((1, 50), lambda i: (0, 0)),
            pl.BlockSpec((50, 50), lambda i: (0, 0)),
            pl.BlockSpec((50, 50), lambda i: (0, 0)),
            pl.BlockSpec((1, 50), lambda i: (0, 0)),
            pl.BlockSpec((50, pa), lambda i: (0, 0)),
            pl.BlockSpec((1, pa), lambda i: (0, 0)),
            pl.BlockSpec((50, ATOM_NUM), lambda i: (0, 0)),
            pl.BlockSpec((1, ATOM_NUM), lambda i: (0, 0)),
        ],
        out_specs=pl.BlockSpec((HB, pa), lambda i: (i, 0)),
        compiler_params=pltpu.CompilerParams(
            dimension_semantics=("parallel",)),
    )(mapf, state, wmf, b_mf[None, :], w_s1, b_s1[None, :],
      w_s2, b_s2[None, :], w_js, w_jm, b_j[None, :],
      wq, bq[None, :], wsv, bsv[None, :])


def _build_s2d4(x, B):
    pf = x[:, STATE_DIM:].reshape(B, 2, 25, 4, 25, 4).astype(jnp.bfloat16)
    return jnp.transpose(pf, (0, 2, 4, 3, 5, 1)).reshape(B, 25, 25, 32)


def _expand_w1(w1c):
    blk = w1c.reshape(3, 3, 8, 4, 32)
    parts = [jnp.pad(blk, ((rh, 1 - rh), (rw, 1 - rw), (0, 0), (0, 0), (0, 0)))
             for rh in range(2) for rw in range(2)]
    w1e = jnp.stack(parts, axis=4)
    w1e = w1e.reshape(128, 512)
    perm = []
    for al in range(2):
        for be in range(2):
            for dhp in range(2):
                for rh in range(2):
                    for dwp in range(2):
                        for rw in range(2):
                            for ic in range(2):
                                perm.append((2 * al + dhp) * 32
                                            + (2 * be + dwp) * 8
                                            + rh * 4 + rw * 2 + ic)
    w1e = w1e[jnp.asarray(perm), :]
    return w1e.astype(jnp.bfloat16)


def kernel(x, w1c, b1, w2c, b2, w_mf_t, b_mf, w_s1, b_s1, w_s2, b_s2,
           w_js, w_jm, b_j, wq, bq, wsv, bsv):
    B = x.shape[0]
    BB = next(bb for bb in (8, 4, 2, 1) if B % bb == 0)
    HB = B // 2 if B % 2 == 0 else B

    state = x[:, :STATE_DIM]
    s2d4 = _build_s2d4(x, B)
    w1e = _expand_w1(w1c)
    b1t = jnp.tile(b1, 4)[None, :]
    w2f = w2c.reshape(3, 384, 128).astype(jnp.bfloat16)
    wmf_p = (w_mf_t.reshape(50, 4, 144, 32).transpose(0, 2, 1, 3)
             .reshape(50, MAP_FULL_DIM).astype(jnp.bfloat16))

    y2p = _convs_call(s2d4, w1e, b1t, w2f, b2[None, :], BB)
    mapf = y2p.reshape(B, MAP_FULL_DIM)
    out = _heads_call(mapf, state, wmf_p, b_mf,
                      w_s1, b_s1, w_s2, b_s2, w_js, w_jm, b_j,
                      wq, bq, wsv, bsv, HB)
    return out.reshape(B, POLICY_DIM, ATOM_NUM)


# speedup vs baseline: 1.0803x; 1.0050x over previous
import jax
import jax.numpy as jnp
from jax import lax
from jax.experimental import pallas as pl
from jax.experimental.pallas import tpu as pltpu

HIGHEST = lax.Precision.HIGHEST

STATE_DIM = 8
POLICY_DIM = 4
ATOM_NUM = 5
S2 = 24
S2P = 26
NP2 = S2 * S2
MAP_FULL_DIM = NP2 * 32


def _convs_kernel(x4_ref, w1_ref, b1_ref, w2_ref, b2_ref, o_ref,
                  p_ref, w3_ref):
    BB = x4_ref.shape[0]
    M = BB * NP2

    for a in range(2):
        for b in range(2):
            p_ref[:, :, :, (2 * a + b) * 32:(2 * a + b + 1) * 32] = \
                x4_ref[:, a:a + 24, b:b + 24, :]

    acc1 = jnp.dot(p_ref[...].reshape(M, 128), w1_ref[...],
                   preferred_element_type=jnp.float32)
    m = jnp.maximum(jnp.maximum(acc1[:, 0:128], acc1[:, 128:256]),
                    jnp.maximum(acc1[:, 256:384], acc1[:, 384:512]))
    y1 = jnp.maximum(m + b1_ref[...], 0.0).astype(jnp.bfloat16)
    y1r = y1.reshape(BB, S2, S2, 128)

    w3_ref[:, :S2, :, 0:128] = y1r
    w3_ref[:, :S2, 0:23, 128:256] = y1r[:, :, 1:24, :]
    w3_ref[:, :S2, 23:24, 128:256] = jnp.zeros((BB, S2, 1, 128), jnp.bfloat16)
    w3_ref[:, :S2, 0:22, 256:384] = y1r[:, :, 2:24, :]
    w3_ref[:, :S2, 22:24, 256:384] = jnp.zeros((BB, S2, 2, 128), jnp.bfloat16)
    w3_ref[:, S2:, :, :] = jnp.zeros((BB, 2, S2, 384), jnp.bfloat16)

    acc = None
    for qh in range(3):
        lhs = w3_ref[:, qh:qh + S2, :, :].reshape(M, 384)
        d = jnp.dot(lhs, w2_ref[qh], preferred_element_type=jnp.float32)
        acc = d if acc is None else acc + d
    m2 = jnp.maximum(acc, pltpu.roll(acc, shift=64, axis=1))
    m2 = jnp.maximum(m2, pltpu.roll(m2, shift=32, axis=1))
    y2 = jnp.maximum(m2[:, :32] + b2_ref[...], 0.0)
    y2 = y2.astype(jnp.bfloat16).reshape(BB, NP2, 32)
    for j in range(4):
        o_ref[:, :, 32 * j:32 * (j + 1)] = y2[:, 144 * j:144 * (j + 1), :]


def _convs_call(s2d4, w1e, b1t, w2f, b2, BB):
    B = s2d4.shape[0]
    return pl.pallas_call(
        _convs_kernel,
        out_shape=jax.ShapeDtypeStruct((B, 144, 128), jnp.bfloat16),
        grid=(B // BB,),
        in_specs=[
            pl.BlockSpec((BB, 25, 25, 32), lambda i: (i, 0, 0, 0)),
            pl.BlockSpec((128, 512), lambda i: (0, 0)),
            pl.BlockSpec((1, 128), lambda i: (0, 0)),
            pl.BlockSpec((3, 384, 128), lambda i: (0, 0, 0)),
            pl.BlockSpec((1, 32), lambda i: (0, 0)),
        ],
        out_specs=pl.BlockSpec((BB, 144, 128), lambda i: (i, 0, 0)),
        scratch_shapes=[
            pltpu.VMEM((BB, S2, S2, 128), jnp.bfloat16),
            pltpu.VMEM((BB, S2P, S2, 384), jnp.bfloat16),
        ],
        compiler_params=pltpu.CompilerParams(
            dimension_semantics=("parallel",)),
    )(s2d4, w1e, b1t, w2f, b2)


def _heads_kernel(mapf_ref, st_ref, wmf_ref, bmf_ref, ws1_ref, bs1_ref,
                  ws2_ref, bs2_ref, wjs_ref, wjm_ref, bj_ref,
                  wq_ref, bq_ref, wsv_ref, bsv_ref, o_ref):
    def dot(a, b):
        return jnp.dot(a, b, precision=HIGHEST,
                       preferred_element_type=jnp.float32)

    map_lat = jnp.maximum(
        lax.dot_general(mapf_ref[...], wmf_ref[...],
                        (((1,), (1,)), ((), ())),
                        preferred_element_type=jnp.float32) + bmf_ref[...],
        0.0)
    h = jnp.maximum(dot(st_ref[...], ws1_ref[...]) + bs1_ref[...], 0.0)
    st_lat = jnp.maximum(dot(h, ws2_ref[...]) + bs2_ref[...], 0.0)
    joint = jnp.maximum(dot(st_lat, wjs_ref[...])
                        + dot(map_lat.astype(jnp.float32), wjm_ref[...])
                        + bj_ref[...], 0.0)
    q = dot(joint, wq_ref[...]) + bq_ref[...]
    sv = dot(joint, wsv_ref[...]) + bsv_ref[...]

    chunks = [q[:, a * ATOM_NUM:(a + 1) * ATOM_NUM] for a in range(POLICY_DIM)]
    qmean = sum(chunks) * (1.0 / POLICY_DIM)
    chunks = [sv + c - qmean for c in chunks]
    outs = []
    for z in chunks:
        mx = jnp.max(z, axis=-1, keepdims=True)
        lse = jnp.log(jnp.sum(jnp.exp(z - mx), axis=-1, keepdims=True)) + mx
        outs.append(z - lse)
    o_ref[...] = jnp.concatenate(outs, axis=-1)


def _heads_call(mapf, state, wmf, b_mf, w_s1, b_s1, w_s2, b_s2,
                w_js, w_jm, b_j, wq, bq, wsv, bsv, HB):
    B, K = mapf.shape
    pa = POLICY_DIM * ATOM_NUM
    return pl.pallas_call(
        _heads_kernel,
        out_shape=jax.ShapeDtypeStruct((B, pa), jnp.float32),
        grid=(B // HB,),
        in_specs=[
            pl.BlockSpec((HB, K), lambda i: (i, 0)),
            pl.BlockSpec((HB, STATE_DIM), lambda i: (i, 0)),
            pl.BlockSpec((50, K), lambda i: (0, 0)),
            pl.BlockSpec((1, 50), lambda i: (0, 0)),
            pl.BlockSpec((STATE_DIM, 64), lambda i: (0, 0)),
            pl.BlockSpec((1, 64), lambda i: (0, 0)),
            pl.BlockSpec((64, 50), lambda i: (0, 0)),
            pl.BlockSpec((1, 50), lambda i: (0, 0)),
            pl.BlockSpec((50, 50), lambda i: (0, 0)),
            pl.BlockSpec((50, 50), lambda i: (0, 0)),
            pl.BlockSpec((1, 50), lambda i: (0, 0)),
            pl.BlockSpec((50, pa), lambda i: (0, 0)),
            pl.BlockSpec((1, pa), lambda i: (0, 0)),
            pl.BlockSpec((50, ATOM_NUM), lambda i: (0, 0)),
            pl.BlockSpec((1, ATOM_NUM), lambda i: (0, 0)),
        ],
        out_specs=pl.BlockSpec((HB, pa), lambda i: (i, 0)),
        compiler_params=pltpu.CompilerParams(
            dimension_semantics=("parallel",)),
    )(mapf, state, wmf, b_mf[None, :], w_s1, b_s1[None, :],
      w_s2, b_s2[None, :], w_js, w_jm, b_j[None, :],
      wq, bq[None, :], wsv, bsv[None, :])


def _build_s2d4(x, B):
    pf = x[:, STATE_DIM:].reshape(B, 2, 25, 4, 25, 4).astype(jnp.bfloat16)
    return jnp.transpose(pf, (0, 2, 4, 3, 5, 1)).reshape(B, 25, 25, 32)


def _expand_w1(w1c):
    blk = w1c.reshape(3, 3, 8, 4, 32)
    parts = [jnp.pad(blk, ((rh, 1 - rh), (rw, 1 - rw), (0, 0), (0, 0), (0, 0)))
             for rh in range(2) for rw in range(2)]
    w1e = jnp.stack(parts, axis=4)
    w1e = w1e.reshape(128, 512)
    perm = []
    for al in range(2):
        for be in range(2):
            for dhp in range(2):
                for rh in range(2):
                    for dwp in range(2):
                        for rw in range(2):
                            for ic in range(2):
                                perm.append((2 * al + dhp) * 32
                                            + (2 * be + dwp) * 8
                                            + rh * 4 + rw * 2 + ic)
    w1e = w1e[jnp.asarray(perm), :]
    return w1e.astype(jnp.bfloat16)


def kernel(x, w1c, b1, w2c, b2, w_mf_t, b_mf, w_s1, b_s1, w_s2, b_s2,
           w_js, w_jm, b_j, wq, bq, wsv, bsv):
    B = x.shape[0]
    BB = next(bb for bb in (16, 8, 4, 2, 1) if B % bb == 0)
    HB = B // 2 if B % 2 == 0 else B

    state = x[:, :STATE_DIM]
    s2d4 = _build_s2d4(x, B)
    w1e = _expand_w1(w1c)
    b1t = jnp.tile(b1, 4)[None, :]
    w2f = w2c.reshape(3, 384, 128).astype(jnp.bfloat16)
    wmf_p = (w_mf_t.reshape(50, 4, 144, 32).transpose(0, 2, 1, 3)
             .reshape(50, MAP_FULL_DIM).astype(jnp.bfloat16))

    y2p = _convs_call(s2d4, w1e, b1t, w2f, b2[None, :], BB)
    mapf = y2p.reshape(B, MAP_FULL_DIM)
    out = _heads_call(mapf, state, wmf_p, b_mf,
                      w_s1, b_s1, w_s2, b_s2, w_js, w_jm, b_j,
                      wq, bq, wsv, bsv, HB)
    return out.reshape(B, POLICY_DIM, ATOM_NUM)


# speedup vs baseline: 1.1362x; 1.0517x over previous
import jax
import jax.numpy as jnp
from jax import lax
from jax.experimental import pallas as pl
from jax.experimental.pallas import tpu as pltpu

HIGHEST = lax.Precision.HIGHEST

STATE_DIM = 8
POLICY_DIM = 4
ATOM_NUM = 5
S2 = 24
S2P = 26
NP2 = S2 * S2
MAP_FULL_DIM = NP2 * 32


def _convs_kernel(x4_ref, w1_ref, b1_ref, w2_ref, b2_ref, o_ref,
                  p_ref, w3_ref):
    BB = x4_ref.shape[0]
    M = BB * NP2

    for a in range(2):
        for b in range(2):
            p_ref[:, :, :, (2 * a + b) * 32:(2 * a + b + 1) * 32] = \
                x4_ref[:, a:a + 24, b:b + 24, 0:32]

    acc1 = jnp.dot(p_ref[...].reshape(M, 128), w1_ref[...],
                   preferred_element_type=jnp.float32)
    m = jnp.maximum(jnp.maximum(acc1[:, 0:128], acc1[:, 128:256]),
                    jnp.maximum(acc1[:, 256:384], acc1[:, 384:512]))
    y1 = jnp.maximum(m + b1_ref[...], 0.0).astype(jnp.bfloat16)
    y1r = y1.reshape(BB, S2, S2, 128)

    w3_ref[:, :S2, :, 0:128] = y1r
    w3_ref[:, :S2, 0:23, 128:256] = y1r[:, :, 1:24, :]
    w3_ref[:, :S2, 23:24, 128:256] = jnp.zeros((BB, S2, 1, 128), jnp.bfloat16)
    w3_ref[:, :S2, 0:22, 256:384] = y1r[:, :, 2:24, :]
    w3_ref[:, :S2, 22:24, 256:384] = jnp.zeros((BB, S2, 2, 128), jnp.bfloat16)
    w3_ref[:, S2:, :, :] = jnp.zeros((BB, 2, S2, 384), jnp.bfloat16)

    acc = None
    for qh in range(3):
        lhs = w3_ref[:, qh:qh + S2, :, :].reshape(M, 384)
        d = jnp.dot(lhs, w2_ref[qh], preferred_element_type=jnp.float32)
        acc = d if acc is None else acc + d
    m2 = jnp.maximum(acc, pltpu.roll(acc, shift=64, axis=1))
    m2 = jnp.maximum(m2, pltpu.roll(m2, shift=32, axis=1))
    y2 = jnp.maximum(m2[:, :32] + b2_ref[...], 0.0)
    y2 = y2.astype(jnp.bfloat16).reshape(BB, NP2, 32)
    for j in range(4):
        o_ref[:, :, 32 * j:32 * (j + 1)] = y2[:, 144 * j:144 * (j + 1), :]


def _convs_call(s2d4, w1e, b1t, w2f, b2, BB):
    B = s2d4.shape[0]
    return pl.pallas_call(
        _convs_kernel,
        out_shape=jax.ShapeDtypeStruct((B, 144, 128), jnp.bfloat16),
        grid=(B // BB,),
        in_specs=[
            pl.BlockSpec((BB, 25, 25, 128), lambda i: (i, 0, 0, 0)),
            pl.BlockSpec((128, 512), lambda i: (0, 0)),
            pl.BlockSpec((1, 128), lambda i: (0, 0)),
            pl.BlockSpec((3, 384, 128), lambda i: (0, 0, 0)),
            pl.BlockSpec((1, 32), lambda i: (0, 0)),
        ],
        out_specs=pl.BlockSpec((BB, 144, 128), lambda i: (i, 0, 0)),
        scratch_shapes=[
            pltpu.VMEM((BB, S2, S2, 128), jnp.bfloat16),
            pltpu.VMEM((BB, S2P, S2, 384), jnp.bfloat16),
        ],
        compiler_params=pltpu.CompilerParams(
            dimension_semantics=("parallel",)),
    )(s2d4, w1e, b1t, w2f, b2)


def _heads_kernel(mapf_ref, st_ref, wmf_ref, bmf_ref, ws1_ref, bs1_ref,
                  ws2_ref, bs2_ref, wjs_ref, wjm_ref, bj_ref,
                  wq_ref, bq_ref, wsv_ref, bsv_ref, o_ref):
    def dot(a, b):
        return jnp.dot(a, b, precision=HIGHEST,
                       preferred_element_type=jnp.float32)

    map_lat = jnp.maximum(
        lax.dot_general(mapf_ref[...], wmf_ref[...],
                        (((1,), (1,)), ((), ())),
                        preferred_element_type=jnp.float32) + bmf_ref[...],
        0.0)
    h = jnp.maximum(dot(st_ref[...], ws1_ref[...]) + bs1_ref[...], 0.0)
    st_lat = jnp.maximum(dot(h, ws2_ref[...]) + bs2_ref[...], 0.0)
    joint = jnp.maximum(dot(st_lat, wjs_ref[...])
                        + dot(map_lat.astype(jnp.float32), wjm_ref[...])
                        + bj_ref[...], 0.0)
    q = dot(joint, wq_ref[...]) + bq_ref[...]
    sv = dot(joint, wsv_ref[...]) + bsv_ref[...]

    chunks = [q[:, a * ATOM_NUM:(a + 1) * ATOM_NUM] for a in range(POLICY_DIM)]
    qmean = sum(chunks) * (1.0 / POLICY_DIM)
    chunks = [sv + c - qmean for c in chunks]
    outs = []
    for z in chunks:
        mx = jnp.max(z, axis=-1, keepdims=True)
        lse = jnp.log(jnp.sum(jnp.exp(z - mx), axis=-1, keepdims=True)) + mx
        outs.append(z - lse)
    o_ref[...] = jnp.concatenate(outs, axis=-1)


def _heads_call(mapf, state, wmf, b_mf, w_s1, b_s1, w_s2, b_s2,
                w_js, w_jm, b_j, wq, bq, wsv, bsv, HB):
    B, K = mapf.shape
    pa = POLICY_DIM * ATOM_NUM
    return pl.pallas_call(
        _heads_kernel,
        out_shape=jax.ShapeDtypeStruct((B, pa), jnp.float32),
        grid=(B // HB,),
        in_specs=[
            pl.BlockSpec((HB, K), lambda i: (i, 0)),
            pl.BlockSpec((HB, STATE_DIM), lambda i: (i, 0)),
            pl.BlockSpec((50, K), lambda i: (0, 0)),
            pl.BlockSpec((1, 50), lambda i: (0, 0)),
            pl.BlockSpec((STATE_DIM, 64), lambda i: (0, 0)),
            pl.BlockSpec((1, 64), lambda i: (0, 0)),
            pl.BlockSpec((64, 50), lambda i: (0, 0)),
            pl.BlockSpec((1, 50), lambda i: (0, 0)),
            pl.BlockSpec((50, 50), lambda i: (0, 0)),
            pl.BlockSpec((50, 50), lambda i: (0, 0)),
            pl.BlockSpec((1, 50), lambda i: (0, 0)),
            pl.BlockSpec((50, pa), lambda i: (0, 0)),
            pl.BlockSpec((1, pa), lambda i: (0, 0)),
            pl.BlockSpec((50, ATOM_NUM), lambda i: (0, 0)),
            pl.BlockSpec((1, ATOM_NUM), lambda i: (0, 0)),
        ],
        out_specs=pl.BlockSpec((HB, pa), lambda i: (i, 0)),
        compiler_params=pltpu.CompilerParams(
            dimension_semantics=("parallel",)),
    )(mapf, state, wmf, b_mf[None, :], w_s1, b_s1[None, :],
      w_s2, b_s2[None, :], w_js, w_jm, b_j[None, :],
      wq, bq[None, :], wsv, bsv[None, :])


def _build_s2d4(x, B):
    pf = x[:, STATE_DIM:].reshape(B, 2, 25, 4, 25, 4).astype(jnp.bfloat16)
    s2d4 = jnp.transpose(pf, (0, 2, 4, 3, 5, 1)).reshape(B, 25, 25, 32)
    return jnp.pad(s2d4, ((0, 0), (0, 0), (0, 0), (0, 96)))


def _expand_w1(w1c):
    blk = w1c.reshape(3, 3, 8, 4, 32)
    parts = [jnp.pad(blk, ((rh, 1 - rh), (rw, 1 - rw), (0, 0), (0, 0), (0, 0)))
             for rh in range(2) for rw in range(2)]
    w1e = jnp.stack(parts, axis=4)
    w1e = w1e.reshape(128, 512)
    perm = []
    for al in range(2):
        for be in range(2):
            for dhp in range(2):
                for rh in range(2):
                    for dwp in range(2):
                        for rw in range(2):
                            for ic in range(2):
                                perm.append((2 * al + dhp) * 32
                                            + (2 * be + dwp) * 8
                                            + rh * 4 + rw * 2 + ic)
    w1e = w1e[jnp.asarray(perm), :]
    return w1e.astype(jnp.bfloat16)


def kernel(x, w1c, b1, w2c, b2, w_mf_t, b_mf, w_s1, b_s1, w_s2, b_s2,
           w_js, w_jm, b_j, wq, bq, wsv, bsv):
    B = x.shape[0]
    BB = next(bb for bb in (16, 8, 4, 2, 1) if B % bb == 0)
    HB = B // 2 if B % 2 == 0 else B

    state = x[:, :STATE_DIM]
    s2d4 = _build_s2d4(x, B)
    w1e = _expand_w1(w1c)
    b1t = jnp.tile(b1, 4)[None, :]
    w2f = w2c.reshape(3, 384, 128).astype(jnp.bfloat16)
    wmf_p = (w_mf_t.reshape(50, 4, 144, 32).transpose(0, 2, 1, 3)
             .reshape(50, MAP_FULL_DIM).astype(jnp.bfloat16))

    y2p = _convs_call(s2d4, w1e, b1t, w2f, b2[None, :], BB)
    mapf = y2p.reshape(B, MAP_FULL_DIM)
    out = _heads_call(mapf, state, wmf_p, b_mf,
                      w_s1, b_s1, w_s2, b_s2, w_js, w_jm, b_j,
                      wq, bq, wsv, bsv, HB)
    return out.reshape(B, POLICY_DIM, ATOM_NUM)


# speedup vs baseline: 1.1390x; 1.0025x over previous
import jax
import jax.numpy as jnp
from jax import lax
from jax.experimental import pallas as pl
from jax.experimental.pallas import tpu as pltpu

HIGHEST = lax.Precision.HIGHEST

STATE_DIM = 8
POLICY_DIM = 4
ATOM_NUM = 5
S2 = 24
S2P = 26
NP2 = S2 * S2
MAP_FULL_DIM = NP2 * 32


def _convs_kernel(x4_ref, w1_ref, b1_ref, w2_ref, b2_ref, o_ref,
                  px_ref, p_ref, w3_ref):
    BB = x4_ref.shape[0]
    M = BB * NP2

    px_ref[:, :, :, 0:32] = x4_ref[:, :, 0:24, 0:32]
    px_ref[:, :, :, 32:64] = x4_ref[:, :, 1:25, 0:32]
    p_ref[:, :, :, 0:64] = px_ref[:, 0:24, :, :]
    p_ref[:, :, :, 64:128] = px_ref[:, 1:25, :, :]

    acc1 = jnp.dot(p_ref[...].reshape(M, 128), w1_ref[...],
                   preferred_element_type=jnp.float32)
    m = jnp.maximum(jnp.maximum(acc1[:, 0:128], acc1[:, 128:256]),
                    jnp.maximum(acc1[:, 256:384], acc1[:, 384:512]))
    y1 = jnp.maximum(m + b1_ref[...], 0.0).astype(jnp.bfloat16)
    y1r = y1.reshape(BB, S2, S2, 128)

    w3_ref[:, :S2, :, 0:128] = y1r
    w3_ref[:, :S2, 0:23, 128:256] = y1r[:, :, 1:24, :]
    w3_ref[:, :S2, 23:24, 128:256] = jnp.zeros((BB, S2, 1, 128), jnp.bfloat16)
    w3_ref[:, :S2, 0:22, 256:384] = y1r[:, :, 2:24, :]
    w3_ref[:, :S2, 22:24, 256:384] = jnp.zeros((BB, S2, 2, 128), jnp.bfloat16)
    w3_ref[:, S2:, :, :] = jnp.zeros((BB, 2, S2, 384), jnp.bfloat16)

    acc = None
    for qh in range(3):
        lhs = w3_ref[:, qh:qh + S2, :, :].reshape(M, 384)
        d = jnp.dot(lhs, w2_ref[qh], preferred_element_type=jnp.float32)
        acc = d if acc is None else acc + d
    m2 = jnp.maximum(acc, pltpu.roll(acc, shift=64, axis=1))
    m2 = jnp.maximum(m2, pltpu.roll(m2, shift=32, axis=1))
    y2 = jnp.maximum(m2[:, :32] + b2_ref[...], 0.0)
    y2 = y2.astype(jnp.bfloat16).reshape(BB, NP2, 32)
    for j in range(4):
        o_ref[:, :, 32 * j:32 * (j + 1)] = y2[:, 144 * j:144 * (j + 1), :]


def _convs_call(s2d4, w1e, b1t, w2f, b2, BB):
    B = s2d4.shape[0]
    return pl.pallas_call(
        _convs_kernel,
        out_shape=jax.ShapeDtypeStruct((B, 144, 128), jnp.bfloat16),
        grid=(B // BB,),
        in_specs=[
            pl.BlockSpec((BB, 25, 25, 128), lambda i: (i, 0, 0, 0)),
            pl.BlockSpec((128, 512), lambda i: (0, 0)),
            pl.BlockSpec((1, 128), lambda i: (0, 0)),
            pl.BlockSpec((3, 384, 128), lambda i: (0, 0, 0)),
            pl.BlockSpec((1, 32), lambda i: (0, 0)),
        ],
        out_specs=pl.BlockSpec((BB, 144, 128), lambda i: (i, 0, 0)),
        scratch_shapes=[
            pltpu.VMEM((BB, 25, S2, 64), jnp.bfloat16),
            pltpu.VMEM((BB, S2, S2, 128), jnp.bfloat16),
            pltpu.VMEM((BB, S2P, S2, 384), jnp.bfloat16),
        ],
        compiler_params=pltpu.CompilerParams(
            dimension_semantics=("parallel",)),
    )(s2d4, w1e, b1t, w2f, b2)


def _heads_kernel(mapf_ref, st_ref, wmf_ref, bmf_ref, ws1_ref, bs1_ref,
                  ws2_ref, bs2_ref, wjs_ref, wjm_ref, bj_ref,
                  wq_ref, bq_ref, wsv_ref, bsv_ref, o_ref):
    def dot(a, b):
        return jnp.dot(a, b, precision=HIGHEST,
                       preferred_element_type=jnp.float32)

    map_lat = jnp.maximum(
        lax.dot_general(mapf_ref[...], wmf_ref[...],
                        (((1,), (1,)), ((), ())),
                        preferred_element_type=jnp.float32) + bmf_ref[...],
        0.0)
    h = jnp.maximum(dot(st_ref[...], ws1_ref[...]) + bs1_ref[...], 0.0)
    st_lat = jnp.maximum(dot(h, ws2_ref[...]) + bs2_ref[...], 0.0)
    joint = jnp.maximum(dot(st_lat, wjs_ref[...])
                        + dot(map_lat.astype(jnp.float32), wjm_ref[...])
                        + bj_ref[...], 0.0)
    q = dot(joint, wq_ref[...]) + bq_ref[...]
    sv = dot(joint, wsv_ref[...]) + bsv_ref[...]

    chunks = [q[:, a * ATOM_NUM:(a + 1) * ATOM_NUM] for a in range(POLICY_DIM)]
    qmean = sum(chunks) * (1.0 / POLICY_DIM)
    chunks = [sv + c - qmean for c in chunks]
    outs = []
    for z in chunks:
        mx = jnp.max(z, axis=-1, keepdims=True)
        lse = jnp.log(jnp.sum(jnp.exp(z - mx), axis=-1, keepdims=True)) + mx
        outs.append(z - lse)
    o_ref[...] = jnp.concatenate(outs, axis=-1)


def _heads_call(mapf, state, wmf, b_mf, w_s1, b_s1, w_s2, b_s2,
                w_js, w_jm, b_j, wq, bq, wsv, bsv, HB):
    B, K = mapf.shape
    pa = POLICY_DIM * ATOM_NUM
    return pl.pallas_call(
        _heads_kernel,
        out_shape=jax.ShapeDtypeStruct((B, pa), jnp.float32),
        grid=(B // HB,),
        in_specs=[
            pl.BlockSpec((HB, K), lambda i: (i, 0)),
            pl.BlockSpec((HB, STATE_DIM), lambda i: (i, 0)),
            pl.BlockSpec((50, K), lambda i: (0, 0)),
            pl.BlockSpec((1, 50), lambda i: (0, 0)),
            pl.BlockSpec((STATE_DIM, 64), lambda i: (0, 0)),
            pl.BlockSpec((1, 64), lambda i: (0, 0)),
            pl.BlockSpec((64, 50), lambda i: (0, 0)),
            pl.BlockSpec((1, 50), lambda i: (0, 0)),
            pl.BlockSpec((50, 50), lambda i: (0, 0)),
            pl.BlockSpec((50, 50), lambda i: (0, 0)),
            pl.BlockSpec((1, 50), lambda i: (0, 0)),
            pl.BlockSpec((50, pa), lambda i: (0, 0)),
            pl.BlockSpec((1, pa), lambda i: (0, 0)),
            pl.BlockSpec((50, ATOM_NUM), lambda i: (0, 0)),
            pl.BlockSpec((1, ATOM_NUM), lambda i: (0, 0)),
        ],
        out_specs=pl.BlockSpec((HB, pa), lambda i: (i, 0)),
        compiler_params=pltpu.CompilerParams(
            dimension_semantics=("parallel",)),
    )(mapf, state, wmf, b_mf[None, :], w_s1, b_s1[None, :],
      w_s2, b_s2[None, :], w_js, w_jm, b_j[None, :],
      wq, bq[None, :], wsv, bsv[None, :])


def _build_s2d4(x, B):
    pf = x[:, STATE_DIM:].reshape(B, 2, 25, 4, 25, 4).astype(jnp.bfloat16)
    s2d4 = jnp.transpose(pf, (0, 2, 4, 3, 5, 1)).reshape(B, 25, 25, 32)
    return jnp.pad(s2d4, ((0, 0), (0, 0), (0, 0), (0, 96)))


def _expand_w1(w1c):
    blk = w1c.reshape(3, 3, 8, 4, 32)
    parts = [jnp.pad(blk, ((rh, 1 - rh), (rw, 1 - rw), (0, 0), (0, 0), (0, 0)))
             for rh in range(2) for rw in range(2)]
    w1e = jnp.stack(parts, axis=4)
    w1e = w1e.reshape(128, 512)
    perm = []
    for al in range(2):
        for be in range(2):
            for dhp in range(2):
                for rh in range(2):
                    for dwp in range(2):
                        for rw in range(2):
                            for ic in range(2):
                                perm.append((2 * al + dhp) * 32
                                            + (2 * be + dwp) * 8
                                            + rh * 4 + rw * 2 + ic)
    w1e = w1e[jnp.asarray(perm), :]
    return w1e.astype(jnp.bfloat16)


def kernel(x, w1c, b1, w2c, b2, w_mf_t, b_mf, w_s1, b_s1, w_s2, b_s2,
           w_js, w_jm, b_j, wq, bq, wsv, bsv):
    B = x.shape[0]
    BB = next(bb for bb in (16, 8, 4, 2, 1) if B % bb == 0)
    HB = B // 2 if B % 2 == 0 else B

    state = x[:, :STATE_DIM]
    s2d4 = _build_s2d4(x, B)
    w1e = _expand_w1(w1c)
    b1t = jnp.tile(b1, 4)[None, :]
    w2f = w2c.reshape(3, 384, 128).astype(jnp.bfloat16)
    wmf_p = (w_mf_t.reshape(50, 4, 144, 32).transpose(0, 2, 1, 3)
             .reshape(50, MAP_FULL_DIM).astype(jnp.bfloat16))

    y2p = _convs_call(s2d4, w1e, b1t, w2f, b2[None, :], BB)
    mapf = y2p.reshape(B, MAP_FULL_DIM)
    out = _heads_call(mapf, state, wmf_p, b_mf,
                      w_s1, b_s1, w_s2, b_s2, w_js, w_jm, b_j,
                      wq, bq, wsv, bsv, HB)
    return out.reshape(B, POLICY_DIM, ATOM_NUM)


# speedup vs baseline: 1.1707x; 1.0279x over previous
import jax
import jax.numpy as jnp
from jax import lax
from jax.experimental import pallas as pl
from jax.experimental.pallas import tpu as pltpu

HIGHEST = lax.Precision.HIGHEST

STATE_DIM = 8
POLICY_DIM = 4
ATOM_NUM = 5
S2 = 24
S2P = 26
NP2 = S2 * S2
MAP_FULL_DIM = NP2 * 32


def _convs_kernel(x4_ref, w1_ref, b1_ref, w2_ref, b2_ref, o_ref,
                  px_ref, p_ref, w3_ref):
    BB = x4_ref.shape[0]
    M = BB * NP2

    px_ref[:, :, :, 0:32] = x4_ref[:, :, 0:24, 0:32]
    px_ref[:, :, :, 32:64] = x4_ref[:, :, 1:25, 0:32]
    p_ref[:, :, :, 0:64] = px_ref[:, 0:24, :, :]
    p_ref[:, :, :, 64:128] = px_ref[:, 1:25, :, :]

    acc1 = jnp.dot(p_ref[...].reshape(M, 128), w1_ref[...],
                   preferred_element_type=jnp.float32)
    m = jnp.maximum(jnp.maximum(acc1[:, 0:128], acc1[:, 128:256]),
                    jnp.maximum(acc1[:, 256:384], acc1[:, 384:512]))
    y1 = jnp.maximum(m + b1_ref[...], 0.0).astype(jnp.bfloat16)
    y1r = y1.reshape(BB, S2, S2, 128)

    w3_ref[:, :S2, :, 0:128] = y1r
    w3_ref[:, :S2, 0:23, 128:256] = y1r[:, :, 1:24, :]
    w3_ref[:, :S2, 23:24, 128:256] = jnp.zeros((BB, S2, 1, 128), jnp.bfloat16)
    w3_ref[:, :S2, 0:22, 256:384] = y1r[:, :, 2:24, :]
    w3_ref[:, :S2, 22:24, 256:384] = jnp.zeros((BB, S2, 2, 128), jnp.bfloat16)
    w3_ref[:, S2:, :, :] = jnp.zeros((BB, 2, S2, 384), jnp.bfloat16)

    acc = None
    for qh in range(3):
        lhs = w3_ref[:, qh:qh + S2, :, :].reshape(M, 384)
        d = jnp.dot(lhs, w2_ref[qh], preferred_element_type=jnp.float32)
        acc = d if acc is None else acc + d
    m2 = jnp.maximum(acc, pltpu.roll(acc, shift=64, axis=1))
    m2 = jnp.maximum(m2, pltpu.roll(m2, shift=32, axis=1))
    y2 = jnp.maximum(m2[:, :32] + b2_ref[...], 0.0)
    y2 = y2.astype(jnp.bfloat16).reshape(BB, NP2, 32)
    for j in range(4):
        o_ref[:, :, 32 * j:32 * (j + 1)] = y2[:, 144 * j:144 * (j + 1), :]


def _convs_call(s2d4, w1e, b1t, w2f, b2, BB):
    B = s2d4.shape[0]
    return pl.pallas_call(
        _convs_kernel,
        out_shape=jax.ShapeDtypeStruct((B, 144, 128), jnp.bfloat16),
        grid=(B // BB,),
        in_specs=[
            pl.BlockSpec((BB, 25, 25, 128), lambda i: (i, 0, 0, 0)),
            pl.BlockSpec((128, 512), lambda i: (0, 0)),
            pl.BlockSpec((1, 128), lambda i: (0, 0)),
            pl.BlockSpec((3, 384, 128), lambda i: (0, 0, 0)),
            pl.BlockSpec((1, 32), lambda i: (0, 0)),
        ],
        out_specs=pl.BlockSpec((BB, 144, 128), lambda i: (i, 0, 0)),
        scratch_shapes=[
            pltpu.VMEM((BB, 25, S2, 64), jnp.bfloat16),
            pltpu.VMEM((BB, S2, S2, 128), jnp.bfloat16),
            pltpu.VMEM((BB, S2P, S2, 384), jnp.bfloat16),
        ],
        compiler_params=pltpu.CompilerParams(
            dimension_semantics=("parallel",)),
    )(s2d4, w1e, b1t, w2f, b2)


def _heads_kernel(mapf_ref, st_ref, wmf_ref, bmf_ref, ws1_ref, bs1_ref,
                  ws2_ref, bs2_ref, wjs_ref, wjm_ref, bj_ref,
                  wq_ref, bq_ref, wsv_ref, bsv_ref, o_ref):
    def dot(a, b):
        return jnp.dot(a, b, precision=HIGHEST,
                       preferred_element_type=jnp.float32)

    map_lat = jnp.maximum(
        lax.dot_general(mapf_ref[...].reshape(mapf_ref.shape[0], -1),
                        wmf_ref[...],
                        (((1,), (1,)), ((), ())),
                        preferred_element_type=jnp.float32) + bmf_ref[...],
        0.0)
    h = jnp.maximum(dot(st_ref[...], ws1_ref[...]) + bs1_ref[...], 0.0)
    st_lat = jnp.maximum(dot(h, ws2_ref[...]) + bs2_ref[...], 0.0)
    joint = jnp.maximum(dot(st_lat, wjs_ref[...])
                        + dot(map_lat.astype(jnp.float32), wjm_ref[...])
                        + bj_ref[...], 0.0)
    q = dot(joint, wq_ref[...]) + bq_ref[...]
    sv = dot(joint, wsv_ref[...]) + bsv_ref[...]

    chunks = [q[:, a * ATOM_NUM:(a + 1) * ATOM_NUM] for a in range(POLICY_DIM)]
    qmean = sum(chunks) * (1.0 / POLICY_DIM)
    chunks = [sv + c - qmean for c in chunks]
    outs = []
    for z in chunks:
        mx = jnp.max(z, axis=-1, keepdims=True)
        lse = jnp.log(jnp.sum(jnp.exp(z - mx), axis=-1, keepdims=True)) + mx
        outs.append(z - lse)
    o_ref[...] = jnp.concatenate(outs, axis=-1)


def _heads_call(mapf, state, wmf, b_mf, w_s1, b_s1, w_s2, b_s2,
                w_js, w_jm, b_j, wq, bq, wsv, bsv, HB):
    B = mapf.shape[0]
    K = MAP_FULL_DIM
    pa = POLICY_DIM * ATOM_NUM
    return pl.pallas_call(
        _heads_kernel,
        out_shape=jax.ShapeDtypeStruct((B, pa), jnp.float32),
        grid=(B // HB,),
        in_specs=[
            pl.BlockSpec((HB, 144, 128), lambda i: (i, 0, 0)),
            pl.BlockSpec((HB, STATE_DIM), lambda i: (i, 0)),
            pl.BlockSpec((50, K), lambda i: (0, 0)),
            pl.BlockSpec((1, 50), lambda i: (0, 0)),
            pl.BlockSpec((STATE_DIM, 64), lambda i: (0, 0)),
            pl.BlockSpec((1, 64), lambda i: (0, 0)),
            pl.BlockSpec((64, 50), lambda i: (0, 0)),
            pl.BlockSpec((1, 50), lambda i: (0, 0)),
            pl.BlockSpec((50, 50), lambda i: (0, 0)),
            pl.BlockSpec((50, 50), lambda i: (0, 0)),
            pl.BlockSpec((1, 50), lambda i: (0, 0)),
            pl.BlockSpec((50, pa), lambda i: (0, 0)),
            pl.BlockSpec((1, pa), lambda i: (0, 0)),
            pl.BlockSpec((50, ATOM_NUM), lambda i: (0, 0)),
            pl.BlockSpec((1, ATOM_NUM), lambda i: (0, 0)),
        ],
        out_specs=pl.BlockSpec((HB, pa), lambda i: (i, 0)),
        compiler_params=pltpu.CompilerParams(
            dimension_semantics=("parallel",)),
    )(mapf, state, wmf, b_mf[None, :], w_s1, b_s1[None, :],
      w_s2, b_s2[None, :], w_js, w_jm, b_j[None, :],
      wq, bq[None, :], wsv, bsv[None, :])


def _build_s2d4(x, B):
    pf = x[:, STATE_DIM:].reshape(B, 2, 25, 4, 25, 4).astype(jnp.bfloat16)
    s2d4 = jnp.transpose(pf, (0, 2, 4, 3, 5, 1)).reshape(B, 25, 25, 32)
    return jnp.pad(s2d4, ((0, 0), (0, 0), (0, 0), (0, 96)))


def _expand_w1(w1c):
    blk = w1c.reshape(3, 3, 8, 4, 32)
    parts = [jnp.pad(blk, ((rh, 1 - rh), (rw, 1 - rw), (0, 0), (0, 0), (0, 0)))
             for rh in range(2) for rw in range(2)]
    w1e = jnp.stack(parts, axis=4)
    w1e = w1e.reshape(128, 512)
    perm = []
    for al in range(2):
        for be in range(2):
            for dhp in range(2):
                for rh in range(2):
                    for dwp in range(2):
                        for rw in range(2):
                            for ic in range(2):
                                perm.append((2 * al + dhp) * 32
                                            + (2 * be + dwp) * 8
                                            + rh * 4 + rw * 2 + ic)
    w1e = w1e[jnp.asarray(perm), :]
    return w1e.astype(jnp.bfloat16)


def kernel(x, w1c, b1, w2c, b2, w_mf_t, b_mf, w_s1, b_s1, w_s2, b_s2,
           w_js, w_jm, b_j, wq, bq, wsv, bsv):
    B = x.shape[0]
    BB = next(bb for bb in (16, 8, 4, 2, 1) if B % bb == 0)
    HB = B // 2 if B % 2 == 0 else B

    state = x[:, :STATE_DIM]
    s2d4 = _build_s2d4(x, B)
    w1e = _expand_w1(w1c)
    b1t = jnp.tile(b1, 4)[None, :]
    w2f = w2c.reshape(3, 384, 128).astype(jnp.bfloat16)
    wmf_p = (w_mf_t.reshape(50, 4, 144, 32).transpose(0, 2, 1, 3)
             .reshape(50, MAP_FULL_DIM).astype(jnp.bfloat16))

    y2p = _convs_call(s2d4, w1e, b1t, w2f, b2[None, :], BB)
    out = _heads_call(y2p, state, wmf_p, b_mf,
                      w_s1, b_s1, w_s2, b_s2, w_js, w_jm, b_j,
                      wq, bq, wsv, bsv, HB)
    return out.reshape(B, POLICY_DIM, ATOM_NUM)


# speedup vs baseline: 1.2372x; 1.0567x over previous
import jax
import jax.numpy as jnp
from jax import lax
from jax.experimental import pallas as pl
from jax.experimental.pallas import tpu as pltpu

HIGHEST = lax.Precision.HIGHEST

STATE_DIM = 8
POLICY_DIM = 4
ATOM_NUM = 5
S2 = 24
S2P = 26
NP2 = S2 * S2
MAP_FULL_DIM = NP2 * 32


def _convs_kernel(x4_ref, w1_ref, b1_ref, w2_ref, b2_ref, o_ref,
                  px_ref, p_ref, w3_ref):
    BB = x4_ref.shape[0]
    M = BB * NP2

    x4r = x4_ref[...].reshape(BB, 25, 25, 32)
    px_ref[:, :, :, 0:32] = x4r[:, :, 0:24, :]
    px_ref[:, :, :, 32:64] = x4r[:, :, 1:25, :]
    p_ref[:, :, :, 0:64] = px_ref[:, 0:24, :, :]
    p_ref[:, :, :, 64:128] = px_ref[:, 1:25, :, :]

    acc1 = jnp.dot(p_ref[...].reshape(M, 128), w1_ref[...],
                   preferred_element_type=jnp.float32)
    m = jnp.maximum(jnp.maximum(acc1[:, 0:128], acc1[:, 128:256]),
                    jnp.maximum(acc1[:, 256:384], acc1[:, 384:512]))
    y1 = jnp.maximum(m + b1_ref[...], 0.0).astype(jnp.bfloat16)
    y1r = y1.reshape(BB, S2, S2, 128)

    w3_ref[:, :S2, :, 0:128] = y1r
    w3_ref[:, :S2, 0:23, 128:256] = y1r[:, :, 1:24, :]
    w3_ref[:, :S2, 23:24, 128:256] = jnp.zeros((BB, S2, 1, 128), jnp.bfloat16)
    w3_ref[:, :S2, 0:22, 256:384] = y1r[:, :, 2:24, :]
    w3_ref[:, :S2, 22:24, 256:384] = jnp.zeros((BB, S2, 2, 128), jnp.bfloat16)
    w3_ref[:, S2:, :, :] = jnp.zeros((BB, 2, S2, 384), jnp.bfloat16)

    acc = None
    for qh in range(3):
        lhs = w3_ref[:, qh:qh + S2, :, :].reshape(M, 384)
        d = jnp.dot(lhs, w2_ref[qh], preferred_element_type=jnp.float32)
        acc = d if acc is None else acc + d
    m2 = jnp.maximum(acc, pltpu.roll(acc, shift=64, axis=1))
    m2 = jnp.maximum(m2, pltpu.roll(m2, shift=32, axis=1))
    y2 = jnp.maximum(m2[:, :32] + b2_ref[...], 0.0)
    y2 = y2.astype(jnp.bfloat16).reshape(BB, NP2, 32)
    for j in range(4):
        o_ref[:, :, 32 * j:32 * (j + 1)] = y2[:, 144 * j:144 * (j + 1), :]


def _convs_call(s2d4, w1e, b1t, w2f, b2, BB):
    B = s2d4.shape[0]
    return pl.pallas_call(
        _convs_kernel,
        out_shape=jax.ShapeDtypeStruct((B, 144, 128), jnp.bfloat16),
        grid=(B // BB,),
        in_specs=[
            pl.BlockSpec((BB, 25, 800), lambda i: (i, 0, 0)),
            pl.BlockSpec((128, 512), lambda i: (0, 0)),
            pl.BlockSpec((1, 128), lambda i: (0, 0)),
            pl.BlockSpec((3, 384, 128), lambda i: (0, 0, 0)),
            pl.BlockSpec((1, 32), lambda i: (0, 0)),
        ],
        out_specs=pl.BlockSpec((BB, 144, 128), lambda i: (i, 0, 0)),
        scratch_shapes=[
            pltpu.VMEM((BB, 25, S2, 64), jnp.bfloat16),
            pltpu.VMEM((BB, S2, S2, 128), jnp.bfloat16),
            pltpu.VMEM((BB, S2P, S2, 384), jnp.bfloat16),
        ],
        compiler_params=pltpu.CompilerParams(
            dimension_semantics=("parallel",)),
    )(s2d4, w1e, b1t, w2f, b2)


def _heads_kernel(mapf_ref, st_ref, wmf_ref, bmf_ref, ws1_ref, bs1_ref,
                  ws2_ref, bs2_ref, wjs_ref, wjm_ref, bj_ref,
                  wq_ref, bq_ref, wsv_ref, bsv_ref, o_ref):
    def dot(a, b):
        return jnp.dot(a, b, precision=HIGHEST,
                       preferred_element_type=jnp.float32)

    map_lat = jnp.maximum(
        lax.dot_general(mapf_ref[...].reshape(mapf_ref.shape[0], -1),
                        wmf_ref[...],
                        (((1,), (1,)), ((), ())),
                        preferred_element_type=jnp.float32) + bmf_ref[...],
        0.0)
    h = jnp.maximum(dot(st_ref[...], ws1_ref[...]) + bs1_ref[...], 0.0)
    st_lat = jnp.maximum(dot(h, ws2_ref[...]) + bs2_ref[...], 0.0)
    joint = jnp.maximum(dot(st_lat, wjs_ref[...])
                        + dot(map_lat.astype(jnp.float32), wjm_ref[...])
                        + bj_ref[...], 0.0)
    q = dot(joint, wq_ref[...]) + bq_ref[...]
    sv = dot(joint, wsv_ref[...]) + bsv_ref[...]

    chunks = [q[:, a * ATOM_NUM:(a + 1) * ATOM_NUM] for a in range(POLICY_DIM)]
    qmean = sum(chunks) * (1.0 / POLICY_DIM)
    chunks = [sv + c - qmean for c in chunks]
    outs = []
    for z in chunks:
        mx = jnp.max(z, axis=-1, keepdims=True)
        lse = jnp.log(jnp.sum(jnp.exp(z - mx), axis=-1, keepdims=True)) + mx
        outs.append(z - lse)
    o_ref[...] = jnp.concatenate(outs, axis=-1)


def _heads_call(mapf, state, wmf, b_mf, w_s1, b_s1, w_s2, b_s2,
                w_js, w_jm, b_j, wq, bq, wsv, bsv, HB):
    B = mapf.shape[0]
    K = MAP_FULL_DIM
    pa = POLICY_DIM * ATOM_NUM
    return pl.pallas_call(
        _heads_kernel,
        out_shape=jax.ShapeDtypeStruct((B, pa), jnp.float32),
        grid=(B // HB,),
        in_specs=[
            pl.BlockSpec((HB, 144, 128), lambda i: (i, 0, 0)),
            pl.BlockSpec((HB, STATE_DIM), lambda i: (i, 0)),
            pl.BlockSpec((50, K), lambda i: (0, 0)),
            pl.BlockSpec((1, 50), lambda i: (0, 0)),
            pl.BlockSpec((STATE_DIM, 64), lambda i: (0, 0)),
            pl.BlockSpec((1, 64), lambda i: (0, 0)),
            pl.BlockSpec((64, 50), lambda i: (0, 0)),
            pl.BlockSpec((1, 50), lambda i: (0, 0)),
            pl.BlockSpec((50, 50), lambda i: (0, 0)),
            pl.BlockSpec((50, 50), lambda i: (0, 0)),
            pl.BlockSpec((1, 50), lambda i: (0, 0)),
            pl.BlockSpec((50, pa), lambda i: (0, 0)),
            pl.BlockSpec((1, pa), lambda i: (0, 0)),
            pl.BlockSpec((50, ATOM_NUM), lambda i: (0, 0)),
            pl.BlockSpec((1, ATOM_NUM), lambda i: (0, 0)),
        ],
        out_specs=pl.BlockSpec((HB, pa), lambda i: (i, 0)),
        compiler_params=pltpu.CompilerParams(
            dimension_semantics=("parallel",)),
    )(mapf, state, wmf, b_mf[None, :], w_s1, b_s1[None, :],
      w_s2, b_s2[None, :], w_js, w_jm, b_j[None, :],
      wq, bq[None, :], wsv, bsv[None, :])


def _build_s2d4(x, B):
    pf = x[:, STATE_DIM:].reshape(B, 2, 25, 4, 25, 4).astype(jnp.bfloat16)
    return jnp.transpose(pf, (0, 2, 4, 3, 5, 1)).reshape(B, 25, 800)


def _expand_w1(w1c):
    blk = w1c.reshape(3, 3, 8, 4, 32)
    parts = [jnp.pad(blk, ((rh, 1 - rh), (rw, 1 - rw), (0, 0), (0, 0), (0, 0)))
             for rh in range(2) for rw in range(2)]
    w1e = jnp.stack(parts, axis=4)
    w1e = w1e.reshape(128, 512)
    perm = []
    for al in range(2):
        for be in range(2):
            for dhp in range(2):
                for rh in range(2):
                    for dwp in range(2):
                        for rw in range(2):
                            for ic in range(2):
                                perm.append((2 * al + dhp) * 32
                                            + (2 * be + dwp) * 8
                                            + rh * 4 + rw * 2 + ic)
    w1e = w1e[jnp.asarray(perm), :]
    return w1e.astype(jnp.bfloat16)


def kernel(x, w1c, b1, w2c, b2, w_mf_t, b_mf, w_s1, b_s1, w_s2, b_s2,
           w_js, w_jm, b_j, wq, bq, wsv, bsv):
    B = x.shape[0]
    BB = next(bb for bb in (16, 8, 4, 2, 1) if B % bb == 0)
    HB = B // 2 if B % 2 == 0 else B

    state = x[:, :STATE_DIM]
    s2d4 = _build_s2d4(x, B)
    w1e = _expand_w1(w1c)
    b1t = jnp.tile(b1, 4)[None, :]
    w2f = w2c.reshape(3, 384, 128).astype(jnp.bfloat16)
    wmf_p = (w_mf_t.reshape(50, 4, 144, 32).transpose(0, 2, 1, 3)
             .reshape(50, MAP_FULL_DIM).astype(jnp.bfloat16))

    y2p = _convs_call(s2d4, w1e, b1t, w2f, b2[None, :], BB)
    out = _heads_call(y2p, state, wmf_p, b_mf,
                      w_s1, b_s1, w_s2, b_s2, w_js, w_jm, b_j,
                      wq, bq, wsv, bsv, HB)
    return out.reshape(B, POLICY_DIM, ATOM_NUM)
